# baseline (speedup 1.0000x reference)
import functools

import jax
import jax.numpy as jnp
from jax import lax
from jax.experimental import pallas as pl
from jax.experimental.pallas import tpu as pltpu
from jax.experimental.pallas import tpu_sc as plsc

BATCH_SIZE = 16384
EMBED_DIM = 32

_info = plsc.get_sparse_core_info()
_NC, _NS, _LANES = _info.num_cores, _info.num_subcores, _info.num_lanes
_NW = _NC * _NS
_BPW = BATCH_SIZE // _NW
_GROUPS = _BPW // 16
_RING = 8
_NCH = _BPW // 128


def _vperm(x, idx):
    return lax.gather(
        x,
        idx[:, None],
        lax.GatherDimensionNumbers(
            offset_dims=(), collapsed_slice_dims=(0,), start_index_map=(0,)),
        (1,),
        mode=lax.GatherScatterMode.PROMISE_IN_BOUNDS,
    )


def _schedule(c_sorted):
    ct = c_sorted.reshape(_NW, _BPW)
    blk = ct >> 7
    base = ct & jnp.int32(-128)
    first = jnp.concatenate(
        [jnp.ones((_NW, 1), jnp.bool_), blk[:, 1:] != blk[:, :-1]], axis=1)
    last = jnp.concatenate(
        [blk[:, :-1] != blk[:, 1:], jnp.ones((_NW, 1), jnp.bool_)], axis=1)
    bnum = jnp.cumsum(first.astype(jnp.int32), axis=1) - 1
    dcount = bnum[:, -1:] + 1
    rows = jnp.broadcast_to(
        jnp.arange(_NW, dtype=jnp.int32)[:, None], (_NW, _BPW))
    dlist = jnp.zeros((_NW, _BPW), jnp.int32).at[rows, bnum].set(base)
    tgt = bnum + _RING
    fire_ok = last & (tgt < dcount)
    fire_base = jnp.where(
        fire_ok, jnp.take_along_axis(dlist, jnp.minimum(tgt, _BPW - 1), 1), -1)
    kidx = jnp.arange(16, dtype=jnp.int32)[None, :]
    prime = jnp.where(kidx < jnp.minimum(dcount, _RING),
                      jnp.pad(dlist[:, :_RING], ((0, 0), (0, 16 - _RING))), -1)
    slot = bnum % _RING
    return (first.astype(jnp.int32).reshape(-1), fire_base.reshape(-1),
            slot.reshape(-1), prime)


def _fire(tblT, biasT, ring, ring_b, slot, basecol, sem):
    bc = pl.multiple_of(basecol, 128)
    pltpu.async_copy(tblT.at[:, pl.ds(bc, 128)], ring.at[slot], sem)
    pltpu.async_copy(biasT.at[:, pl.ds(bc, 128)], ring_b.at[slot], sem)


def _wait_pair(tblT, biasT, ring, ring_b, slot, sem):
    pltpu.make_async_copy(tblT.at[:, pl.ds(0, 128)], ring.at[slot], sem).wait()
    pltpu.make_async_copy(biasT.at[:, pl.ds(0, 128)], ring_b.at[slot],
                          sem).wait()


def _phase_u_body(sc_hbm, flag_hbm, fbase_hbm, slot_hbm, prime_hbm, perm_hbm,
                  uembT, ubT, inter_hbm,
                  sc_v, flag_v, fbase_v, slot_v, prime_v, perm_v,
                  ring_u, ring_ub, stg_v, sem, ssem):
    wid = lax.axis_index("s") * _NC + lax.axis_index("c")
    base = wid * _BPW
    pltpu.sync_copy(sc_hbm.at[pl.ds(base, _BPW)], sc_v)
    pltpu.sync_copy(flag_hbm.at[pl.ds(base, _BPW)], flag_v)
    pltpu.sync_copy(fbase_hbm.at[pl.ds(base, _BPW)], fbase_v)
    pltpu.sync_copy(slot_hbm.at[pl.ds(base, _BPW)], slot_v)
    pltpu.sync_copy(prime_hbm.at[wid], prime_v)
    pltpu.sync_copy(perm_hbm.at[wid], perm_v)

    pvec = prime_v[pl.ds(0, 16)]
    for k in range(_RING):
        pb = pvec[k]
        @pl.when(pb >= 0)
        def _():
            _fire(uembT, ubT, ring_u, ring_ub, k, pb, sem)

    iota = lax.iota(jnp.int32, _LANES)
    iota_lo = iota
    iota_hi = iota + 16
    zero16 = jnp.zeros((_LANES,), jnp.int32)

    def group(g, carry):
        b0 = g * 16
        ch = g >> 3
        buf = ch & 1
        r0 = (g & 7) * 16

        @pl.when(((g & 7) == 0) & (g >= 16))
        def _():
            pltpu.make_async_copy(
                inter_hbm.at[pl.ds(0, 128)], stg_v.at[buf], ssem).wait()

        svec = sc_v[pl.ds(b0, 16)]
        fvec = flag_v[pl.ds(b0, 16)]
        bvec = fbase_v[pl.ds(b0, 16)]
        tvec = slot_v[pl.ds(b0, 16)]
        for k in range(16):
            sk = tvec[k]
            @pl.when(fvec[k] != 0)
            def _():
                _wait_pair(uembT, ubT, ring_u, ring_ub, sk, sem)
            lu = jnp.full((_LANES,), svec[k] & 127, jnp.int32)
            u0 = plsc.load_gather(ring_u.at[sk], [iota_lo, lu])
            u1 = plsc.load_gather(ring_u.at[sk], [iota_hi, lu])
            ubv = plsc.load_gather(ring_ub.at[sk], [zero16, lu])
            r = r0 + k
            stg_v[buf, r, pl.ds(0, 16)] = u0
            stg_v[buf, r, pl.ds(16, 16)] = u1
            stg_v[buf, r, pl.ds(32, 16)] = ubv
            fb = bvec[k]
            @pl.when(fb >= 0)
            def _():
                _fire(uembT, ubT, ring_u, ring_ub, sk, fb, sem)

        @pl.when((g & 7) == 7)
        def _():
            pltpu.async_copy(
                stg_v.at[buf], inter_hbm.at[perm_v.at[ch]], ssem)
        return carry

    lax.fori_loop(0, _GROUPS, group, 0)

    for buf in range(2):
        pltpu.make_async_copy(
            inter_hbm.at[pl.ds(0, 128)], stg_v.at[buf], ssem).wait()


def _phase_m_body(sc_hbm, flag_hbm, fbase_hbm, slot_hbm, prime_hbm, perm_hbm,
                  membT, mbT, inter_hbm, out_hbm,
                  sc_v, flag_v, fbase_v, slot_v, prime_v, perm_v,
                  ring_m, ring_mb, ig_v, out_v, sem, isem, osem):
    wid = lax.axis_index("s") * _NC + lax.axis_index("c")
    base = wid * _BPW
    pltpu.sync_copy(sc_hbm.at[pl.ds(base, _BPW)], sc_v)
    pltpu.sync_copy(flag_hbm.at[pl.ds(base, _BPW)], flag_v)
    pltpu.sync_copy(fbase_hbm.at[pl.ds(base, _BPW)], fbase_v)
    pltpu.sync_copy(slot_hbm.at[pl.ds(base, _BPW)], slot_v)
    pltpu.sync_copy(prime_hbm.at[wid], prime_v)
    pltpu.sync_copy(perm_hbm.at[wid], perm_v)

    pvec = prime_v[pl.ds(0, 16)]
    for k in range(_RING):
        pb = pvec[k]
        @pl.when(pb >= 0)
        def _():
            _fire(membT, mbT, ring_m, ring_mb, k, pb, sem)
    pltpu.async_copy(inter_hbm.at[perm_v.at[0]], ig_v.at[0], isem)

    iota = lax.iota(jnp.int32, _LANES)
    iota_lo = iota
    iota_hi = iota + 16
    perms = [iota ^ sh for sh in (8, 4, 2, 1)]
    zero16 = jnp.zeros((_LANES,), jnp.int32)

    def group(g, carry):
        b0 = g * 16
        ch = g >> 3
        buf = ch & 1
        r0 = (g & 7) * 16

        @pl.when((g & 7) == 0)
        def _():
            @pl.when(ch < _NCH - 1)
            def _():
                pltpu.async_copy(
                    inter_hbm.at[perm_v.at[jnp.minimum(ch + 1, _NCH - 1)]],
                    ig_v.at[1 - buf], isem)
            pltpu.make_async_copy(
                inter_hbm.at[pl.ds(0, 128)], ig_v.at[buf], isem).wait()

        svec = sc_v[pl.ds(b0, 16)]
        fvec = flag_v[pl.ds(b0, 16)]
        bvec = fbase_v[pl.ds(b0, 16)]
        tvec = slot_v[pl.ds(b0, 16)]
        acc = jnp.zeros((_LANES,), jnp.float32)
        for k in range(16):
            sk = tvec[k]
            @pl.when(fvec[k] != 0)
            def _():
                _wait_pair(membT, mbT, ring_m, ring_mb, sk, sem)
            lm = jnp.full((_LANES,), svec[k] & 127, jnp.int32)
            m0 = plsc.load_gather(ring_m.at[sk], [iota_lo, lm])
            m1 = plsc.load_gather(ring_m.at[sk], [iota_hi, lm])
            mbv = plsc.load_gather(ring_mb.at[sk], [zero16, lm])
            r = r0 + k
            u0 = ig_v[buf, r, pl.ds(0, 16)]
            u1 = ig_v[buf, r, pl.ds(16, 16)]
            ubv = ig_v[buf, r, pl.ds(32, 16)]
            p = u0 * m0 + u1 * m1
            for pm in perms:
                p = p + _vperm(p, pm)
            acc = jnp.where(iota == k, p + ubv + mbv, acc)
            fb = bvec[k]
            @pl.when(fb >= 0)
            def _():
                _fire(membT, mbT, ring_m, ring_mb, sk, fb, sem)
        out_v[pl.ds(b0, 16)] = acc

        @pl.when((g & 7) == 7)
        def _():
            o = pl.multiple_of(ch * 128, 128)
            pltpu.async_copy(
                out_v.at[pl.ds(o, 128)], out_hbm.at[perm_v.at[ch]], osem)
        return carry

    lax.fori_loop(0, _GROUPS, group, 0)

    for _ in range(_NCH):
        pltpu.make_async_copy(
            out_hbm.at[pl.ds(0, 128)], out_v.at[pl.ds(0, 128)], osem).wait()


_IDX_SCRATCH = [
    pltpu.VMEM((_BPW,), jnp.int32),
    pltpu.VMEM((_BPW,), jnp.int32),
    pltpu.VMEM((_BPW,), jnp.int32),
    pltpu.VMEM((_BPW,), jnp.int32),
    pltpu.VMEM((16,), jnp.int32),
    pltpu.VMEM((_NCH, 128), jnp.int32),
]

_phase_u = functools.partial(
    pl.kernel,
    out_type=jax.ShapeDtypeStruct((BATCH_SIZE, 128), jnp.float32),
    mesh=plsc.VectorSubcoreMesh(core_axis_name="c", subcore_axis_name="s"),
    compiler_params=pltpu.CompilerParams(
        use_tc_tiling_on_sc=True, needs_layout_passes=False),
    scratch_types=_IDX_SCRATCH + [
        pltpu.VMEM((_RING, EMBED_DIM, 128), jnp.float32),
        pltpu.VMEM((_RING, 1, 128), jnp.float32),
        pltpu.VMEM((2, 128, 128), jnp.float32),
        pltpu.SemaphoreType.DMA,
        pltpu.SemaphoreType.DMA,
    ],
)(_phase_u_body)

_phase_m = functools.partial(
    pl.kernel,
    out_type=jax.ShapeDtypeStruct((BATCH_SIZE,), jnp.float32),
    mesh=plsc.VectorSubcoreMesh(core_axis_name="c", subcore_axis_name="s"),
    compiler_params=pltpu.CompilerParams(
        use_tc_tiling_on_sc=True, needs_layout_passes=False),
    scratch_types=_IDX_SCRATCH + [
        pltpu.VMEM((_RING, EMBED_DIM, 128), jnp.float32),
        pltpu.VMEM((_RING, 1, 128), jnp.float32),
        pltpu.VMEM((2, 128, 128), jnp.float32),
        pltpu.VMEM((_BPW,), jnp.float32),
        pltpu.SemaphoreType.DMA,
        pltpu.SemaphoreType.DMA,
        pltpu.SemaphoreType.DMA,
    ],
)(_phase_m_body)


def kernel(inputs, user_emb, movie_emb, user_bias, movie_bias):
    idx = inputs.astype(jnp.int32)
    uid = idx[:, 0]
    mid = idx[:, 1]

    order_u = jnp.arange(BATCH_SIZE, dtype=jnp.int32)
    su = jnp.take(uid, order_u)
    flag_u, fbase_u, slot_u, prime_u = _schedule(su)
    perm_u = order_u.reshape(_NW, _NCH, 128)

    order_m = jnp.arange(BATCH_SIZE, dtype=jnp.int32)
    sm = jnp.take(mid, order_m)
    flag_m, fbase_m, slot_m, prime_m = _schedule(sm)
    perm_m = order_m.reshape(_NW, _NCH, 128)

    inter = _phase_u(su, flag_u, fbase_u, slot_u, prime_u, perm_u,
                     user_emb.T, user_bias.T)
    return _phase_m(sm, flag_m, fbase_m, slot_m, prime_m, perm_m,
                    movie_emb.T, movie_bias.T, inter)


# speedup vs baseline: 2.1216x; 2.1216x over previous
import functools

import jax
import jax.numpy as jnp
from jax import lax
from jax.experimental import pallas as pl
from jax.experimental.pallas import tpu as pltpu
from jax.experimental.pallas import tpu_sc as plsc

BATCH_SIZE = 16384
EMBED_DIM = 32
NUM_ROWS = 1000000

_info = plsc.get_sparse_core_info()
_NC, _NS, _LANES = _info.num_cores, _info.num_subcores, _info.num_lanes
_NW = _NC * _NS
_BPW = BATCH_SIZE // _NW
_CHUNK = 128
_NCHUNK = _BPW // _CHUNK
_GROUPS = _BPW // 16
_RING = 8
_STAGE = 142848


def _vperm(x, idx):
    return lax.gather(
        x,
        idx[:, None],
        lax.GatherDimensionNumbers(
            offset_dims=(), collapsed_slice_dims=(0,), start_index_map=(0,)),
        (1,),
        mode=lax.GatherScatterMode.PROMISE_IN_BOUNDS,
    )


def _sc_body(uid_hbm, mid_hbm, uembT_hbm, membT_hbm, ub_hbm, mb_hbm,
             out_hbm, uid_v, mid_v, ring_u, ring_m, ring_mb, ub_sh,
             out_v, sem, bias_sem):
    cid = lax.axis_index("c")
    sid = lax.axis_index("s")
    wid = sid * _NC + cid
    base = wid * _BPW

    pltpu.sync_copy(uid_hbm.at[pl.ds(base, _BPW)], uid_v)
    pltpu.sync_copy(mid_hbm.at[pl.ds(base, _BPW)], mid_v)

    def fire(cu, cm, slot):
        bu = pl.multiple_of(cu & jnp.int32(-128), 128)
        bm = pl.multiple_of(cm & jnp.int32(-128), 128)
        pltpu.async_copy(uembT_hbm.at[:, pl.ds(bu, 128)], ring_u.at[slot], sem)
        pltpu.async_copy(membT_hbm.at[:, pl.ds(bm, 128)], ring_m.at[slot], sem)
        pltpu.async_copy(mb_hbm.at[:, pl.ds(bm, 128)], ring_mb.at[slot], sem)

    cvec_u0 = uid_v[pl.ds(0, 16)]
    cvec_m0 = mid_v[pl.ds(0, 16)]
    for k in range(_RING):
        fire(cvec_u0[k], cvec_m0[k], k)

    half = jnp.where(sid < 8, sid, sid - 8)
    off = pl.multiple_of(jnp.where(half < 7, half * _STAGE, NUM_ROWS - 128),
                         128)

    @pl.when(sid < 8)
    def _():
        @pl.when(half < 7)
        def _():
            pltpu.sync_copy(ub_hbm.at[0, pl.ds(off, _STAGE)],
                            ub_sh.at[pl.ds(off, _STAGE)])
        @pl.when(half == 7)
        def _():
            pltpu.sync_copy(ub_hbm.at[0, pl.ds(off, 128)],
                            ub_sh.at[pl.ds(off, 128)])

    plsc.subcore_barrier()

    bias_copies = []
    for c in range(_NCHUNK):
        s = pl.ds(c * _CHUNK, _CHUNK)
        bias_copies.append(
            pltpu.async_copy(ub_sh.at[uid_v.at[s]], out_v.at[s], bias_sem))
    for cp in bias_copies:
        cp.wait()

    iota = lax.iota(jnp.int32, _LANES)
    perms = [iota ^ sh for sh in (8, 4, 2, 1)]
    iota_lo = iota
    iota_hi = iota + 16

    def group(g, carry):
        b0 = g * 16
        gnext = jnp.minimum(g + 1, _GROUPS - 1)
        cvec_u = uid_v[pl.ds(b0, 16)]
        cvec_m = mid_v[pl.ds(b0, 16)]
        cnext_u = uid_v[pl.ds(gnext * 16, 16)]
        cnext_m = mid_v[pl.ds(gnext * 16, 16)]
        acc = jnp.zeros((_LANES,), jnp.float32)
        for k in range(16):
            slot = k % _RING
            pltpu.make_async_copy(
                uembT_hbm.at[:, pl.ds(0, 128)], ring_u.at[slot], sem).wait()
            pltpu.make_async_copy(
                membT_hbm.at[:, pl.ds(0, 128)], ring_m.at[slot], sem).wait()
            pltpu.make_async_copy(
                mb_hbm.at[:, pl.ds(0, 128)], ring_mb.at[slot], sem).wait()
            lu = jnp.full((_LANES,), cvec_u[k] & 127, jnp.int32)
            lm = jnp.full((_LANES,), cvec_m[k] & 127, jnp.int32)
            u0 = plsc.load_gather(ring_u.at[slot], [iota_lo, lu])
            u1 = plsc.load_gather(ring_u.at[slot], [iota_hi, lu])
            m0 = plsc.load_gather(ring_m.at[slot], [iota_lo, lm])
            m1 = plsc.load_gather(ring_m.at[slot], [iota_hi, lm])
            mbv = plsc.load_gather(ring_mb.at[slot], [jnp.zeros((_LANES,), jnp.int32), lm])
            p = u0 * m0 + u1 * m1
            for pm in perms:
                p = p + _vperm(p, pm)
            acc = jnp.where(iota == k, p + mbv, acc)
            if k < 16 - _RING:
                fire(cvec_u[k + _RING], cvec_m[k + _RING], slot)
            else:
                @pl.when(g < _GROUPS - 1)
                def _():
                    fire(cnext_u[k + _RING - 16], cnext_m[k + _RING - 16], slot)
        out_v[pl.ds(b0, 16)] = acc + out_v[pl.ds(b0, 16)]
        return carry

    lax.fori_loop(0, _GROUPS, group, 0)

    pltpu.sync_copy(out_v, out_hbm.at[pl.ds(base, _BPW)])


_sc_kernel = functools.partial(
    pl.kernel,
    out_type=jax.ShapeDtypeStruct((BATCH_SIZE,), jnp.float32),
    mesh=plsc.VectorSubcoreMesh(core_axis_name="c", subcore_axis_name="s"),
    compiler_params=pltpu.CompilerParams(
        use_tc_tiling_on_sc=True, needs_layout_passes=False),
    scratch_types=[
        pltpu.VMEM((_BPW,), jnp.int32),
        pltpu.VMEM((_BPW,), jnp.int32),
        pltpu.VMEM((_RING, EMBED_DIM, 128), jnp.float32),
        pltpu.VMEM((_RING, EMBED_DIM, 128), jnp.float32),
        pltpu.VMEM((_RING, 1, 128), jnp.float32),
        pltpu.VMEM_SHARED((NUM_ROWS,), jnp.float32),
        pltpu.VMEM((_BPW,), jnp.float32),
        pltpu.SemaphoreType.DMA,
        pltpu.SemaphoreType.DMA,
    ],
)(_sc_body)


def kernel(inputs, user_emb, movie_emb, user_bias, movie_bias):
    idx = inputs.astype(jnp.int32)
    uid = idx[:, 0]
    mid = idx[:, 1]
    return _sc_kernel(uid, mid, user_emb.T, movie_emb.T,
                      user_bias.T, movie_bias.T)
